# baseline (speedup 1.0000x reference)
import functools

import jax
import jax.numpy as jnp
from jax import lax
from jax.experimental import pallas as pl
from jax.experimental.pallas import tpu as pltpu
from jax.experimental.pallas import tpu_sc as plsc

B = 1024
K = 1000
V = 2048
VSUB = 600
NKP = 16
NBB = 8

NC = 2
NS = 16
NW = NC * NS
QPW = B // NW
L = 16

DV = V * 3
DS = VSUB * 3
DSM = 144

_EPS = 1e-6


def _coeff_body(ang_ref, go_ref, tr_ref, out_ref):
    a = ang_ref[:, 0:1]
    ca = jnp.cos(0.5 * a)
    sa = jnp.sin(0.5 * a)
    w2 = ca
    z2 = -sa

    g = go_ref[...]
    gx = g[:, 0:1]
    gy = g[:, 1:2]
    gz = g[:, 2:3]
    n2 = gx * gx + gy * gy + gz * gz
    n = jnp.sqrt(n2)
    w1 = jnp.cos(0.5 * n)
    safe = jnp.where(n < _EPS, jnp.ones_like(n), n)
    s = jnp.where(n < _EPS, 0.5 - n2 / 48.0, jnp.sin(0.5 * n) / safe)
    x1 = gx * s
    y1 = gy * s
    z1 = gz * s

    wt = w1 * w2 - z1 * z2
    xt = x1 * w2 + y1 * z2
    yt = y1 * w2 - x1 * z2
    zt = w1 * z2 + z1 * w2

    def rot(w, x, y, z):
        two = jnp.float32(2.0)
        one = jnp.float32(1.0)
        return [
            one - two * (y * y + z * z), two * (x * y - w * z), two * (x * z + w * y),
            two * (x * y + w * z), one - two * (x * x + z * z), two * (y * z - w * x),
            two * (x * z - w * y), two * (y * z + w * x), one - two * (x * x + y * y),
        ]

    t = tr_ref[...]
    cols = rot(wt, xt, yt, zt) + rot(w1, x1, y1, z1) + [t[:, 0:1], t[:, 1:2], t[:, 2:3]]
    out_ref[...] = jnp.concatenate(
        cols + [jnp.zeros((B, 11), jnp.float32)], axis=1)


def _make_coeffs(angles, global_orient, transl):
    return pl.pallas_call(
        _coeff_body,
        out_shape=jax.ShapeDtypeStruct((B, 32), jnp.float32),
    )(angles, global_orient, transl)


CQ = 8
NCHUNK = QPW // CQ


OFF_PS = DS
OFF_SM = DS + VSUB
OFF_DIA = DS + VSUB + DSM
OFF_PB = DS + VSUB + DSM + 1
DC = 2688


def _sc_body(v_hbm, c_hbm, co_hbm, qi_hbm,
             ov_hbm, os_hbm, ob_hbm, ok_hbm, od_hbm,
             qi_v, co_v, dov,
             vin, cin,
             vout, svout, bbout, kpout,
             sem, sem_v, sem_s, sem_b, sem_k):
    wid = lax.axis_index("s") * NC + lax.axis_index("c")
    base = wid * QPW

    iota = lax.iota(jnp.int32, L)
    iota3 = iota * 3
    zero16 = jnp.zeros((L,), jnp.int32)
    lane0 = zero16
    lane1 = zero16 + 1
    lane2 = zero16 + 2

    pltpu.sync_copy(qi_hbm.at[pl.ds(base, QPW)], qi_v)
    pltpu.sync_copy(co_hbm.at[pl.ds(base, QPW)], co_v)

    def splat(val):
        return zero16 + val

    def per_chunk(c, carry0):
        idx8 = qi_v.at[pl.ds(c * CQ, CQ)]
        cv = pltpu.async_copy(v_hbm.at[idx8], vin, sem)
        cc8 = pltpu.async_copy(c_hbm.at[idx8], cin, sem)
        cv.wait()
        cc8.wait()

        def per_query(q, carry):
            i = c * CQ + q
            b = base + i
            sq = splat(q)
            par = jnp.bitwise_and(i, 1)
            spar = splat(par)
            vo = vout.at[pl.ds(par, 1)]
            so = svout.at[pl.ds(par, 1)]
            bo_ = bbout.at[pl.ds(par, 1)]
            ko = kpout.at[pl.ds(par, 1)]

            @pl.when(i >= 2)
            def _drain():
                pltpu.make_async_copy(vo, ov_hbm.at[pl.ds(b - 2, 1)], sem_v).wait()
                pltpu.make_async_copy(so, os_hbm.at[pl.ds(b - 2, 1)], sem_s).wait()
                pltpu.make_async_copy(bo_, ob_hbm.at[pl.ds(b - 2, 1)], sem_b).wait()
                pltpu.make_async_copy(ko, ok_hbm.at[pl.ds(b - 2, 1)], sem_k).wait()

            si = splat(i)
            cc = [plsc.load_gather(co_v, [si, splat(k)]) for k in range(21)]
            mt = cc[0:9]
            mb = cc[9:18]
            tx, ty, tz = cc[18], cc[19], cc[20]

            def apply16(x, y, z, sel):
                me = [jnp.where(sel, a, b) for (a, b) in zip(mt, mb)]
                ox = me[0] * x + me[1] * y + me[2] * z + tx
                oy = me[3] * x + me[4] * y + me[5] * z + ty
                oz = me[6] * x + me[7] * y + me[8] * z + tz
                return ox, oy, oz

            def vstep(j, _):
                vidx = iota + j * L
                bx = iota3 + j * (3 * L)
                x = plsc.load_gather(vin, [sq, bx])
                y = plsc.load_gather(vin, [sq, bx + 1])
                z = plsc.load_gather(vin, [sq, bx + 2])
                wf = plsc.load_gather(cin, [sq, (vidx >> 5) + OFF_PB])
                wi = plsc.bitcast(wf, jnp.int32)
                pbit = (wi >> (vidx & 31)) & 1
                ox, oy, oz = apply16(x, y, z, pbit == 1)
                plsc.store_scatter(vout, [spar, bx], ox)
                plsc.store_scatter(vout, [spar, bx + 1], oy)
                plsc.store_scatter(vout, [spar, bx + 2], oz)
                return 0

            lax.fori_loop(0, V // L, vstep, 0, unroll=False)

            def sstep(j, _):
                vidx = iota + j * L
                bx = iota3 + j * (3 * L)
                x = plsc.load_gather(cin, [sq, bx])
                y = plsc.load_gather(cin, [sq, bx + 1])
                z = plsc.load_gather(cin, [sq, bx + 2])
                pvf = plsc.load_gather(cin, [sq, vidx + OFF_PS])
                pv = plsc.bitcast(pvf, jnp.int32)
                ox, oy, oz = apply16(x, y, z, pv == 1)
                plsc.store_scatter(svout, [spar, bx], ox)
                plsc.store_scatter(svout, [spar, bx + 1], oy)
                plsc.store_scatter(svout, [spar, bx + 2], oz)
                return 0

            nfull = VSUB // L
            lax.fori_loop(0, nfull, sstep, 0, unroll=False)

            ntail = VSUB - nfull * L
            mtail = iota < ntail
            bx = iota3 + nfull * (3 * L)
            bxc = jnp.minimum(bx, DS - 3)
            x = plsc.load_gather(cin, [sq, bxc], mask=mtail)
            y = plsc.load_gather(cin, [sq, bxc + 1], mask=mtail)
            z = plsc.load_gather(cin, [sq, bxc + 2], mask=mtail)
            pvc = jnp.minimum(iota + nfull * L, VSUB - 1) + OFF_PS
            pvf = plsc.load_gather(cin, [sq, pvc], mask=mtail)
            pv = plsc.bitcast(pvf, jnp.int32)
            ox, oy, oz = apply16(x, y, z, pv == 1)
            plsc.store_scatter(svout, [spar, bxc], ox, mask=mtail)
            plsc.store_scatter(svout, [spar, bxc + 1], oy, mask=mtail)
            plsc.store_scatter(svout, [spar, bxc + 2], oz, mask=mtail)

            for (src0, dref, dst0, sel) in (
                    (0, bbout, 0, iota < NBB),
                    (48, kpout, 0, iota >= 0),
                    (96, kpout, 48, iota < 0),
            ):
                bx = iota3 + (OFF_SM + src0)
                x = plsc.load_gather(cin, [sq, bx])
                y = plsc.load_gather(cin, [sq, bx + 1])
                z = plsc.load_gather(cin, [sq, bx + 2])
                ox, oy, oz = apply16(x, y, z, sel)
                bo = iota3 + dst0
                plsc.store_scatter(dref, [spar, bo], ox)
                plsc.store_scatter(dref, [spar, bo + 1], oy)
                plsc.store_scatter(dref, [spar, bo + 2], oz)

            dvec = plsc.load_gather(cin, [sq, splat(OFF_DIA)])
            plsc.store_scatter(dov, [splat(i)], dvec, mask=iota < 1)

            pltpu.async_copy(vo, ov_hbm.at[pl.ds(b, 1)], sem_v)
            pltpu.async_copy(so, os_hbm.at[pl.ds(b, 1)], sem_s)
            pltpu.async_copy(bo_, ob_hbm.at[pl.ds(b, 1)], sem_b)
            pltpu.async_copy(ko, ok_hbm.at[pl.ds(b, 1)], sem_k)
            return 0

        lax.fori_loop(0, CQ, per_query, 0, unroll=False)
        return 0

    lax.fori_loop(0, NCHUNK, per_chunk, 0, unroll=False)

    for par in (0, 1):
        b2 = base + QPW - 2 + par
        pltpu.make_async_copy(
            vout.at[pl.ds(par, 1)], ov_hbm.at[pl.ds(b2, 1)], sem_v).wait()
        pltpu.make_async_copy(
            svout.at[pl.ds(par, 1)], os_hbm.at[pl.ds(b2, 1)], sem_s).wait()
        pltpu.make_async_copy(
            bbout.at[pl.ds(par, 1)], ob_hbm.at[pl.ds(b2, 1)], sem_b).wait()
        pltpu.make_async_copy(
            kpout.at[pl.ds(par, 1)], ok_hbm.at[pl.ds(b2, 1)], sem_k).wait()

    pltpu.sync_copy(dov, od_hbm.at[pl.ds(base, QPW)])


def _sc_call(v2, comb, coeffs, qidx):
    mesh = plsc.VectorSubcoreMesh(
        core_axis_name="c", subcore_axis_name="s", num_cores=NC,
        num_subcores=NS)
    f = pl.kernel(
        _sc_body,
        out_type=(
            jax.ShapeDtypeStruct((B, DV), jnp.float32),
            jax.ShapeDtypeStruct((B, DS), jnp.float32),
            jax.ShapeDtypeStruct((B, 48), jnp.float32),
            jax.ShapeDtypeStruct((B, 96), jnp.float32),
            jax.ShapeDtypeStruct((B,), jnp.float32),
        ),
        mesh=mesh,
        compiler_params=pltpu.CompilerParams(needs_layout_passes=False, use_tc_tiling_on_sc=True),
        scratch_types=[
            pltpu.VMEM((QPW,), jnp.int32),
            pltpu.VMEM((QPW, 32), jnp.float32),
            pltpu.VMEM((QPW,), jnp.float32),
            pltpu.VMEM((CQ, DV), jnp.float32),
            pltpu.VMEM((CQ, DC), jnp.float32),
            pltpu.VMEM((2, DV), jnp.float32),
            pltpu.VMEM((2, DS), jnp.float32),
            pltpu.VMEM((2, 48), jnp.float32),
            pltpu.VMEM((2, 96), jnp.float32),
            pltpu.SemaphoreType.DMA,
            pltpu.SemaphoreType.DMA,
            pltpu.SemaphoreType.DMA,
            pltpu.SemaphoreType.DMA,
            pltpu.SemaphoreType.DMA,
        ],
    )
    return f(v2, comb, coeffs, qidx)


def kernel(angles, global_orient, transl, query_idx, v_table, v_sub_table,
           bbox_top_table, bbox_bottom_table, kp_top_table, kp_bottom_table,
           diameter_table, parts_table, parts_sub_table):
    coeffs = _make_coeffs(angles, global_orient, transl)

    qidx = query_idx.astype(jnp.int32)
    v2 = v_table.reshape(K, DV)
    ps_bits = jax.lax.bitcast_convert_type(
        parts_sub_table.astype(jnp.int32), jnp.float32)
    lsb = (parts_table.astype(jnp.uint32) & jnp.uint32(1)).reshape(K, V // 32, 32)
    pbits = jnp.sum(
        lsb << jnp.arange(32, dtype=jnp.uint32)[None, None, :], axis=2,
        dtype=jnp.uint32)
    pbits_f = jax.lax.bitcast_convert_type(pbits.astype(jnp.int32), jnp.float32)
    comb = jnp.concatenate([
        v_sub_table.reshape(K, DS),
        ps_bits,
        bbox_top_table.reshape(K, NBB * 3),
        bbox_bottom_table.reshape(K, NBB * 3),
        kp_top_table.reshape(K, NKP * 3),
        kp_bottom_table.reshape(K, NKP * 3),
        diameter_table[:, None],
        pbits_f,
        jnp.zeros((K, DC - OFF_PB - V // 32), jnp.float32),
    ], axis=1)

    ov, os_, ob, ok, od = _sc_call(v2, comb, coeffs, qidx)

    return (
        ov.reshape(B, V, 3),
        os_.reshape(B, VSUB, 3),
        ob.reshape(B, 2 * NBB, 3),
        ok.reshape(B, 2 * NKP, 3),
        od,
    )


# speedup vs baseline: 1.0970x; 1.0970x over previous
import functools

import jax
import jax.numpy as jnp
from jax import lax
from jax.experimental import pallas as pl
from jax.experimental.pallas import tpu as pltpu
from jax.experimental.pallas import tpu_sc as plsc

B = 1024
K = 1000
V = 2048
VSUB = 600
NKP = 16
NBB = 8

NC = 2
NS = 16
NW = NC * NS
QPW = B // NW
L = 16

DV = V * 3
DS = VSUB * 3
DSM = 144

_EPS = 1e-6


def _coeff_body(ang_ref, go_ref, tr_ref, out_ref):
    a = ang_ref[:, 0:1]
    ca = jnp.cos(0.5 * a)
    sa = jnp.sin(0.5 * a)
    w2 = ca
    z2 = -sa

    g = go_ref[...]
    gx = g[:, 0:1]
    gy = g[:, 1:2]
    gz = g[:, 2:3]
    n2 = gx * gx + gy * gy + gz * gz
    n = jnp.sqrt(n2)
    w1 = jnp.cos(0.5 * n)
    safe = jnp.where(n < _EPS, jnp.ones_like(n), n)
    s = jnp.where(n < _EPS, 0.5 - n2 / 48.0, jnp.sin(0.5 * n) / safe)
    x1 = gx * s
    y1 = gy * s
    z1 = gz * s

    wt = w1 * w2 - z1 * z2
    xt = x1 * w2 + y1 * z2
    yt = y1 * w2 - x1 * z2
    zt = w1 * z2 + z1 * w2

    def rot(w, x, y, z):
        two = jnp.float32(2.0)
        one = jnp.float32(1.0)
        return [
            one - two * (y * y + z * z), two * (x * y - w * z), two * (x * z + w * y),
            two * (x * y + w * z), one - two * (x * x + z * z), two * (y * z - w * x),
            two * (x * z - w * y), two * (y * z + w * x), one - two * (x * x + y * y),
        ]

    t = tr_ref[...]
    cols = rot(wt, xt, yt, zt) + rot(w1, x1, y1, z1) + [t[:, 0:1], t[:, 1:2], t[:, 2:3]]
    out_ref[...] = jnp.concatenate(
        cols + [jnp.zeros((B, 11), jnp.float32)], axis=1)


def _make_coeffs(angles, global_orient, transl):
    return pl.pallas_call(
        _coeff_body,
        out_shape=jax.ShapeDtypeStruct((B, 32), jnp.float32),
    )(angles, global_orient, transl)


CQ = 8
NCHUNK = QPW // CQ


OFF_PS = DS
OFF_SM = DS + VSUB
OFF_DIA = DS + VSUB + DSM
DC = 2560


def _sc_body(v_hbm, p_hbm, c_hbm, co_hbm, qi_hbm,
             ov_hbm, os_hbm, ob_hbm, ok_hbm, od_hbm,
             qi_v, co_v, dov,
             vin, pin, cin,
             vout, svout, bbout, kpout,
             sem, sem_v, sem_s, sem_b, sem_k):
    wid = lax.axis_index("s") * NC + lax.axis_index("c")
    base = wid * QPW

    iota = lax.iota(jnp.int32, L)
    iota3 = iota * 3
    zero16 = jnp.zeros((L,), jnp.int32)
    lane0 = zero16
    lane1 = zero16 + 1
    lane2 = zero16 + 2

    pltpu.sync_copy(qi_hbm.at[pl.ds(base, QPW)], qi_v)
    pltpu.sync_copy(co_hbm.at[pl.ds(base, QPW)], co_v)

    def splat(val):
        return zero16 + val

    def per_chunk(c, carry0):
        idx8 = qi_v.at[pl.ds(c * CQ, CQ)]
        cv = pltpu.async_copy(v_hbm.at[idx8], vin, sem)
        cp = pltpu.async_copy(p_hbm.at[idx8], pin, sem)
        cc8 = pltpu.async_copy(c_hbm.at[idx8], cin, sem)
        cv.wait()
        cp.wait()
        cc8.wait()

        def per_query(q, carry):
            i = c * CQ + q
            b = base + i
            sq = splat(q)
            par = jnp.bitwise_and(i, 1)
            spar = splat(par)
            vo = vout.at[pl.ds(par, 1)]
            so = svout.at[pl.ds(par, 1)]
            bo_ = bbout.at[pl.ds(par, 1)]
            ko = kpout.at[pl.ds(par, 1)]

            @pl.when(i >= 2)
            def _drain():
                pltpu.make_async_copy(vo, ov_hbm.at[pl.ds(b - 2, 1)], sem_v).wait()
                pltpu.make_async_copy(so, os_hbm.at[pl.ds(b - 2, 1)], sem_s).wait()
                pltpu.make_async_copy(bo_, ob_hbm.at[pl.ds(b - 2, 1)], sem_b).wait()
                pltpu.make_async_copy(ko, ok_hbm.at[pl.ds(b - 2, 1)], sem_k).wait()

            si = splat(i)
            cc = [plsc.load_gather(co_v, [si, splat(k)]) for k in range(21)]
            mt = cc[0:9]
            mb = cc[9:18]
            tx, ty, tz = cc[18], cc[19], cc[20]

            def apply16(x, y, z, sel):
                me = [jnp.where(sel, a, b) for (a, b) in zip(mt, mb)]
                ox = me[0] * x + me[1] * y + me[2] * z + tx
                oy = me[3] * x + me[4] * y + me[5] * z + ty
                oz = me[6] * x + me[7] * y + me[8] * z + tz
                return ox, oy, oz

            def vstep(j, _):
                vidx = iota + j * L
                bx = iota3 + j * (3 * L)
                x = plsc.load_gather(vin, [sq, bx])
                y = plsc.load_gather(vin, [sq, bx + 1])
                z = plsc.load_gather(vin, [sq, bx + 2])
                pv = plsc.load_gather(pin, [sq, vidx])
                ox, oy, oz = apply16(x, y, z, pv == 1)
                plsc.store_scatter(vout, [spar, bx], ox)
                plsc.store_scatter(vout, [spar, bx + 1], oy)
                plsc.store_scatter(vout, [spar, bx + 2], oz)
                return 0

            lax.fori_loop(0, V // L, vstep, 0, unroll=False)

            def sstep(j, _):
                vidx = iota + j * L
                bx = iota3 + j * (3 * L)
                x = plsc.load_gather(cin, [sq, bx])
                y = plsc.load_gather(cin, [sq, bx + 1])
                z = plsc.load_gather(cin, [sq, bx + 2])
                pvf = plsc.load_gather(cin, [sq, vidx + OFF_PS])
                pv = plsc.bitcast(pvf, jnp.int32)
                ox, oy, oz = apply16(x, y, z, pv == 1)
                plsc.store_scatter(svout, [spar, bx], ox)
                plsc.store_scatter(svout, [spar, bx + 1], oy)
                plsc.store_scatter(svout, [spar, bx + 2], oz)
                return 0

            nfull = VSUB // L
            lax.fori_loop(0, nfull, sstep, 0, unroll=False)

            ntail = VSUB - nfull * L
            mtail = iota < ntail
            bx = iota3 + nfull * (3 * L)
            bxc = jnp.minimum(bx, DS - 3)
            x = plsc.load_gather(cin, [sq, bxc], mask=mtail)
            y = plsc.load_gather(cin, [sq, bxc + 1], mask=mtail)
            z = plsc.load_gather(cin, [sq, bxc + 2], mask=mtail)
            pvc = jnp.minimum(iota + nfull * L, VSUB - 1) + OFF_PS
            pvf = plsc.load_gather(cin, [sq, pvc], mask=mtail)
            pv = plsc.bitcast(pvf, jnp.int32)
            ox, oy, oz = apply16(x, y, z, pv == 1)
            plsc.store_scatter(svout, [spar, bxc], ox, mask=mtail)
            plsc.store_scatter(svout, [spar, bxc + 1], oy, mask=mtail)
            plsc.store_scatter(svout, [spar, bxc + 2], oz, mask=mtail)

            for (src0, dref, dst0, sel) in (
                    (0, bbout, 0, iota < NBB),
                    (48, kpout, 0, iota >= 0),
                    (96, kpout, 48, iota < 0),
            ):
                bx = iota3 + (OFF_SM + src0)
                x = plsc.load_gather(cin, [sq, bx])
                y = plsc.load_gather(cin, [sq, bx + 1])
                z = plsc.load_gather(cin, [sq, bx + 2])
                ox, oy, oz = apply16(x, y, z, sel)
                bo = iota3 + dst0
                plsc.store_scatter(dref, [spar, bo], ox)
                plsc.store_scatter(dref, [spar, bo + 1], oy)
                plsc.store_scatter(dref, [spar, bo + 2], oz)

            dvec = plsc.load_gather(cin, [sq, splat(OFF_DIA)])
            plsc.store_scatter(dov, [splat(i)], dvec, mask=iota < 1)

            pltpu.async_copy(vo, ov_hbm.at[pl.ds(b, 1)], sem_v)
            pltpu.async_copy(so, os_hbm.at[pl.ds(b, 1)], sem_s)
            pltpu.async_copy(bo_, ob_hbm.at[pl.ds(b, 1)], sem_b)
            pltpu.async_copy(ko, ok_hbm.at[pl.ds(b, 1)], sem_k)
            return 0

        lax.fori_loop(0, CQ, per_query, 0, unroll=False)
        return 0

    lax.fori_loop(0, NCHUNK, per_chunk, 0, unroll=False)

    for par in (0, 1):
        b2 = base + QPW - 2 + par
        pltpu.make_async_copy(
            vout.at[pl.ds(par, 1)], ov_hbm.at[pl.ds(b2, 1)], sem_v).wait()
        pltpu.make_async_copy(
            svout.at[pl.ds(par, 1)], os_hbm.at[pl.ds(b2, 1)], sem_s).wait()
        pltpu.make_async_copy(
            bbout.at[pl.ds(par, 1)], ob_hbm.at[pl.ds(b2, 1)], sem_b).wait()
        pltpu.make_async_copy(
            kpout.at[pl.ds(par, 1)], ok_hbm.at[pl.ds(b2, 1)], sem_k).wait()

    pltpu.sync_copy(dov, od_hbm.at[pl.ds(base, QPW)])


def _sc_call(v2, p2, comb, coeffs, qidx):
    mesh = plsc.VectorSubcoreMesh(
        core_axis_name="c", subcore_axis_name="s", num_cores=NC,
        num_subcores=NS)
    f = pl.kernel(
        _sc_body,
        out_type=(
            jax.ShapeDtypeStruct((B, DV), jnp.float32),
            jax.ShapeDtypeStruct((B, DS), jnp.float32),
            jax.ShapeDtypeStruct((B, 48), jnp.float32),
            jax.ShapeDtypeStruct((B, 96), jnp.float32),
            jax.ShapeDtypeStruct((B,), jnp.float32),
        ),
        mesh=mesh,
        compiler_params=pltpu.CompilerParams(needs_layout_passes=False, use_tc_tiling_on_sc=True),
        scratch_types=[
            pltpu.VMEM((QPW,), jnp.int32),
            pltpu.VMEM((QPW, 32), jnp.float32),
            pltpu.VMEM((QPW,), jnp.float32),
            pltpu.VMEM((CQ, DV), jnp.float32),
            pltpu.VMEM((CQ, V), jnp.int32),
            pltpu.VMEM((CQ, DC), jnp.float32),
            pltpu.VMEM((2, DV), jnp.float32),
            pltpu.VMEM((2, DS), jnp.float32),
            pltpu.VMEM((2, 48), jnp.float32),
            pltpu.VMEM((2, 96), jnp.float32),
            pltpu.SemaphoreType.DMA,
            pltpu.SemaphoreType.DMA,
            pltpu.SemaphoreType.DMA,
            pltpu.SemaphoreType.DMA,
            pltpu.SemaphoreType.DMA,
        ],
    )
    return f(v2, p2, comb, coeffs, qidx)


def kernel(angles, global_orient, transl, query_idx, v_table, v_sub_table,
           bbox_top_table, bbox_bottom_table, kp_top_table, kp_bottom_table,
           diameter_table, parts_table, parts_sub_table):
    coeffs = _make_coeffs(angles, global_orient, transl)

    qidx = query_idx.astype(jnp.int32)
    v2 = v_table.reshape(K, DV)
    p2 = parts_table.astype(jnp.int32)
    ps_bits = jax.lax.bitcast_convert_type(
        parts_sub_table.astype(jnp.int32), jnp.float32)
    comb = jnp.concatenate([
        v_sub_table.reshape(K, DS),
        ps_bits,
        bbox_top_table.reshape(K, NBB * 3),
        bbox_bottom_table.reshape(K, NBB * 3),
        kp_top_table.reshape(K, NKP * 3),
        kp_bottom_table.reshape(K, NKP * 3),
        diameter_table[:, None],
        jnp.zeros((K, DC - DS - VSUB - DSM - 1), jnp.float32),
    ], axis=1)

    ov, os_, ob, ok, od = _sc_call(v2, p2, comb, coeffs, qidx)

    return (
        ov.reshape(B, V, 3),
        os_.reshape(B, VSUB, 3),
        ob.reshape(B, 2 * NBB, 3),
        ok.reshape(B, 2 * NKP, 3),
        od,
    )
